# Initial kernel scaffold; baseline (speedup 1.0000x reference)
import jax
import jax.numpy as jnp
from jax.experimental import pallas as pl


def kernel(y_true, y_pred):
    raise NotImplementedError("write your pallas kernel here")


# speedup vs baseline: 112.2589x; 112.2589x over previous
import functools

import jax
import jax.numpy as jnp
from jax import lax
from jax.experimental import pallas as pl
from jax.experimental.pallas import tpu as pltpu
from jax.experimental.pallas import tpu_sc as plsc

_IGNORE = -1
_TOPK = 5


_D = 128


def _gather_label_rows(y_true_flat, y_pred_rows):
    info = plsc.get_sparse_core_info()
    nc, ns, L = info.num_cores, info.num_subcores, info.num_lanes
    nw = nc * ns
    B = y_true_flat.shape[0]
    vocab = (y_pred_rows.shape[0] * y_pred_rows.shape[1]) // B
    per_w = B // nw
    mesh = plsc.VectorSubcoreMesh(core_axis_name="c", subcore_axis_name="s")

    @functools.partial(
        pl.kernel,
        mesh=mesh,
        out_type=jax.ShapeDtypeStruct((B, _D), jnp.float32),
        scratch_types=[
            pltpu.VMEM((per_w,), jnp.int32),
            pltpu.VMEM((per_w,), jnp.int32),
            pltpu.VMEM((per_w, _D), jnp.float32),
            pltpu.SemaphoreType.DMA,
        ],
    )
    def k(yt_hbm, yp_hbm, rows_hbm, t_v, idx_v, rows_v, sem):
        wid = lax.axis_index("s") * nc + lax.axis_index("c")
        base = wid * per_w
        pltpu.sync_copy(yt_hbm.at[pl.ds(base, per_w)], t_v)
        for j in range(per_w // L):
            t = jnp.clip(t_v[pl.ds(j * L, L)], 0, vocab - 1)
            rows = base + j * L + lax.iota(jnp.int32, L)
            idx_v[pl.ds(j * L, L)] = lax.shift_right_logical(
                rows * vocab + t, 7)
        pltpu.async_copy(yp_hbm.at[idx_v], rows_v, sem).wait()
        pltpu.sync_copy(rows_v, rows_hbm.at[pl.ds(base, per_w)])

    return k(y_true_flat, y_pred_rows)


def _tc_body(nblocks, vocab, t_ref, rows_ref, yp_ref, out_ref, acc_ref):
    i = pl.program_id(0)

    @pl.when(i == 0)
    def _init():
        acc_ref[0] = 0.0
        acc_ref[1] = 0.0

    x = yp_ref[...]
    tt = t_ref[...]
    rows = rows_ref[...]
    R = rows.shape[0]
    r_global = (i * R
                + lax.broadcasted_iota(jnp.int32, (R, 1), 0))
    lane = jnp.bitwise_and(
        r_global * vocab + jnp.clip(tt, 0, vocab - 1), _D - 1)
    lidx = lax.broadcasted_iota(jnp.int32, rows.shape, 1)
    vv = jnp.sum(jnp.where(lidx == lane, rows, 0.0),
                 axis=1, keepdims=True)
    shape = x.shape
    col = lax.broadcasted_iota(jnp.int32, shape, 1)
    gt = jnp.sum((x > vv).astype(jnp.float32), axis=1, keepdims=True)
    eq_before = jnp.sum(
        jnp.logical_and(x == vv, col < tt).astype(jnp.float32),
        axis=1, keepdims=True)
    valid = tt != _IGNORE
    hit = jnp.logical_and(gt + eq_before < float(_TOPK), valid)
    acc_ref[0] += jnp.sum(hit.astype(jnp.float32))
    acc_ref[1] += jnp.sum(valid.astype(jnp.float32))

    @pl.when(i == nblocks - 1)
    def _fini():
        out_ref[0, 0] = 100.0 * acc_ref[0] / acc_ref[1]


def kernel(y_true, y_pred):
    B = y_true.size
    vocab = y_pred.shape[-1]
    yt = y_true.reshape(B).astype(jnp.int32)
    yp2d = y_pred.reshape(B, vocab)

    label_rows = _gather_label_rows(yt, y_pred.reshape(-1, _D))

    R = 8
    nblocks = B // R
    out = pl.pallas_call(
        functools.partial(_tc_body, nblocks, vocab),
        grid=(nblocks,),
        in_specs=[
            pl.BlockSpec((R, 1), lambda i: (i, 0)),
            pl.BlockSpec((R, _D), lambda i: (i, 0)),
            pl.BlockSpec((R, vocab), lambda i: (i, 0)),
        ],
        out_specs=pl.BlockSpec((1, 1), lambda i: (0, 0),
                               memory_space=pltpu.SMEM),
        out_shape=jax.ShapeDtypeStruct((1, 1), jnp.float32),
        scratch_shapes=[pltpu.SMEM((2,), jnp.float32)],
    )(yt.reshape(B, 1), label_rows, yp2d)
    return out[0, 0]


# speedup vs baseline: 129.5898x; 1.1544x over previous
import functools

import jax
import jax.numpy as jnp
from jax import lax
from jax.experimental import pallas as pl
from jax.experimental.pallas import tpu as pltpu
from jax.experimental.pallas import tpu_sc as plsc

_IGNORE = -1
_TOPK = 5


_D = 128


def _gather_label_rows(y_true_flat, y_pred_rows):
    info = plsc.get_sparse_core_info()
    nc, ns, L = info.num_cores, info.num_subcores, info.num_lanes
    nw = nc * ns
    B = y_true_flat.shape[0]
    vocab = (y_pred_rows.shape[0] * y_pred_rows.shape[1]) // B
    per_w = B // nw
    mesh = plsc.VectorSubcoreMesh(core_axis_name="c", subcore_axis_name="s")

    @functools.partial(
        pl.kernel,
        mesh=mesh,
        out_type=jax.ShapeDtypeStruct((B, _D), jnp.float32),
        scratch_types=[
            pltpu.VMEM((per_w,), jnp.int32),
            pltpu.VMEM((per_w,), jnp.int32),
            pltpu.VMEM((per_w, _D), jnp.float32),
            pltpu.SemaphoreType.DMA,
        ],
    )
    def k(yt_hbm, yp_hbm, rows_hbm, t_v, idx_v, rows_v, sem):
        wid = lax.axis_index("s") * nc + lax.axis_index("c")
        base = wid * per_w
        pltpu.sync_copy(yt_hbm.at[pl.ds(base, per_w)], t_v)
        for j in range(per_w // L):
            t = jnp.clip(t_v[pl.ds(j * L, L)], 0, vocab - 1)
            rows = base + j * L + lax.iota(jnp.int32, L)
            idx_v[pl.ds(j * L, L)] = lax.shift_right_logical(
                rows * vocab + t, 7)
        pltpu.async_copy(yp_hbm.at[idx_v], rows_v, sem).wait()
        pltpu.sync_copy(rows_v, rows_hbm.at[pl.ds(base, per_w)])

    return k(y_true_flat, y_pred_rows)


def _tc_body(nblocks, vocab, t_ref, rows_ref, yp_ref, out_ref, acc_ref):
    i = pl.program_id(0)

    @pl.when(i == 0)
    def _init():
        acc_ref[0] = 0.0
        acc_ref[1] = 0.0

    x = yp_ref[...]
    tt = t_ref[...]
    rows = rows_ref[...]
    R = rows.shape[0]
    r_global = (i * R
                + lax.broadcasted_iota(jnp.int32, (R, 1), 0))
    lane = jnp.bitwise_and(
        r_global * vocab + jnp.clip(tt, 0, vocab - 1), _D - 1)
    lidx = lax.broadcasted_iota(jnp.int32, rows.shape, 1)
    vv = jnp.sum(jnp.where(lidx == lane, rows, 0.0),
                 axis=1, keepdims=True)
    shape = x.shape
    col = lax.broadcasted_iota(jnp.int32, shape, 1)
    beats = jnp.logical_or(
        x > vv, jnp.logical_and(x == vv, col < tt))
    rank = jnp.sum(beats.astype(jnp.float32), axis=1, keepdims=True)
    valid = tt != _IGNORE
    hit = jnp.logical_and(rank < float(_TOPK), valid)
    acc_ref[0] += jnp.sum(hit.astype(jnp.float32))
    acc_ref[1] += jnp.sum(valid.astype(jnp.float32))

    @pl.when(i == nblocks - 1)
    def _fini():
        out_ref[0, 0] = 100.0 * acc_ref[0] / acc_ref[1]


def kernel(y_true, y_pred):
    B = y_true.size
    vocab = y_pred.shape[-1]
    yt = y_true.reshape(B).astype(jnp.int32)
    yp2d = y_pred.reshape(B, vocab)

    label_rows = _gather_label_rows(yt, y_pred.reshape(-1, _D))

    R = 32
    nblocks = B // R
    out = pl.pallas_call(
        functools.partial(_tc_body, nblocks, vocab),
        grid=(nblocks,),
        in_specs=[
            pl.BlockSpec((R, 1), lambda i: (i, 0)),
            pl.BlockSpec((R, _D), lambda i: (i, 0)),
            pl.BlockSpec((R, vocab), lambda i: (i, 0)),
        ],
        out_specs=pl.BlockSpec((1, 1), lambda i: (0, 0),
                               memory_space=pltpu.SMEM),
        out_shape=jax.ShapeDtypeStruct((1, 1), jnp.float32),
        scratch_shapes=[pltpu.SMEM((2,), jnp.float32)],
    )(yt.reshape(B, 1), label_rows, yp2d)
    return out[0, 0]


# speedup vs baseline: 533.1102x; 4.1138x over previous
import functools

import jax
import jax.numpy as jnp
from jax import lax
from jax.experimental import pallas as pl
from jax.experimental.pallas import tpu as pltpu
from jax.experimental.pallas import tpu_sc as plsc

_IGNORE = -1
_TOPK = 5


_D = 128


def _gather_label_rows(y_true_flat, y_pred_rows):
    info = plsc.get_sparse_core_info()
    nc, ns, L = info.num_cores, info.num_subcores, info.num_lanes
    nw = nc * ns
    B = y_true_flat.shape[0]
    vocab = (y_pred_rows.shape[0] * y_pred_rows.shape[1]) // B
    per_w = B // nw
    mesh = plsc.VectorSubcoreMesh(core_axis_name="c", subcore_axis_name="s")

    @functools.partial(
        pl.kernel,
        mesh=mesh,
        out_type=jax.ShapeDtypeStruct((B, _D), jnp.float32),
        scratch_types=[
            pltpu.VMEM((per_w,), jnp.int32),
            pltpu.VMEM((per_w,), jnp.int32),
            pltpu.VMEM((per_w, _D), jnp.float32),
            pltpu.SemaphoreType.DMA,
        ],
    )
    def k(yt_hbm, yp_hbm, rows_hbm, t_v, idx_v, rows_v, sem):
        wid = lax.axis_index("s") * nc + lax.axis_index("c")
        base = wid * per_w
        pltpu.sync_copy(yt_hbm.at[pl.ds(base, per_w)], t_v)
        for j in range(per_w // L):
            t = jnp.clip(t_v[pl.ds(j * L, L)], 0, vocab - 1)
            rows = base + j * L + lax.iota(jnp.int32, L)
            idx_v[pl.ds(j * L, L)] = lax.shift_right_logical(
                rows * vocab + t, 7)
        pltpu.async_copy(yp_hbm.at[idx_v], rows_v, sem).wait()
        pltpu.sync_copy(rows_v, rows_hbm.at[pl.ds(base, per_w)])

    return k(y_true_flat, y_pred_rows)


def _tc_body(nblocks, vocab, t_ref, yp_ref, out_ref, acc_ref):
    i = pl.program_id(0)

    @pl.when(i == 0)
    def _init():
        acc_ref[0] = 0.0
        acc_ref[1] = 0.0

    x = yp_ref[...]
    tt = t_ref[...]
    tcl = jnp.clip(tt, 0, vocab - 1)
    col = lax.broadcasted_iota(jnp.int32, x.shape, 1)
    vv = jnp.sum(jnp.where(col == tcl, x, 0.0),
                 axis=1, keepdims=True)
    beats = jnp.logical_or(
        x > vv, jnp.logical_and(x == vv, col < tcl))
    rank = jnp.sum(beats.astype(jnp.float32), axis=1, keepdims=True)
    valid = tt != _IGNORE
    hit = jnp.logical_and(rank < float(_TOPK), valid)
    acc_ref[0] += jnp.sum(hit.astype(jnp.float32))
    acc_ref[1] += jnp.sum(valid.astype(jnp.float32))

    @pl.when(i == nblocks - 1)
    def _fini():
        out_ref[0, 0] = 100.0 * acc_ref[0] / acc_ref[1]


def kernel(y_true, y_pred):
    B = y_true.size
    vocab = y_pred.shape[-1]
    yt = y_true.reshape(B).astype(jnp.int32)
    yp2d = y_pred.reshape(B, vocab)

    R = 32
    nblocks = B // R
    out = pl.pallas_call(
        functools.partial(_tc_body, nblocks, vocab),
        grid=(nblocks,),
        in_specs=[
            pl.BlockSpec((R, 1), lambda i: (i, 0)),
            pl.BlockSpec((R, vocab), lambda i: (i, 0)),
        ],
        out_specs=pl.BlockSpec((1, 1), lambda i: (0, 0),
                               memory_space=pltpu.SMEM),
        out_shape=jax.ShapeDtypeStruct((1, 1), jnp.float32),
        scratch_shapes=[pltpu.SMEM((2,), jnp.float32)],
    )(yt.reshape(B, 1), yp2d)
    return out[0, 0]
